# baseline (speedup 1.0000x reference)
import functools

import jax
import jax.numpy as jnp
from jax import lax
from jax.experimental import pallas as pl
from jax.experimental.pallas import tpu as pltpu
from jax.experimental.pallas import tpu_sc as plsc

_NC = 2
_NS = 16
_NW = _NC * _NS


def _sc_gather(user_ids, item_ids, user_weight, ub2d, item_weight, ib2d):
    B = user_ids.shape[0]
    D = user_weight.shape[1]
    b_per_w = B // _NW

    mesh = plsc.VectorSubcoreMesh(core_axis_name="c", subcore_axis_name="s")

    @functools.partial(
        pl.kernel,
        mesh=mesh,
        out_type=[
            jax.ShapeDtypeStruct((B, D), jnp.float32),
            jax.ShapeDtypeStruct((B, 1), jnp.float32),
            jax.ShapeDtypeStruct((B, D), jnp.float32),
            jax.ShapeDtypeStruct((B, 1), jnp.float32),
        ],
        scratch_types=[
            pltpu.VMEM((b_per_w,), jnp.int32),
            pltpu.VMEM((b_per_w,), jnp.int32),
            pltpu.VMEM((b_per_w, D), jnp.float32),
            pltpu.VMEM((b_per_w, D), jnp.float32),
            pltpu.VMEM((b_per_w, 1), jnp.float32),
            pltpu.VMEM((b_per_w, 1), jnp.float32),
            pltpu.SemaphoreType.DMA,
        ],
        compiler_params=pltpu.CompilerParams(use_tc_tiling_on_sc=False),
    )
    def gather(uids_hbm, iids_hbm, uw_hbm, ub_hbm, iw_hbm, ib_hbm,
               uw_out, ub_out, iw_out, ib_out,
               uidx_v, iidx_v, uw_v, iw_v, ubg, ibg, sem):
        wid = lax.axis_index("s") * _NC + lax.axis_index("c")
        base = wid * b_per_w
        pltpu.sync_copy(uids_hbm.at[pl.ds(base, b_per_w)], uidx_v)
        pltpu.sync_copy(iids_hbm.at[pl.ds(base, b_per_w)], iidx_v)
        cu = pltpu.async_copy(uw_hbm.at[uidx_v], uw_v, sem)
        ci = pltpu.async_copy(iw_hbm.at[iidx_v], iw_v, sem)
        cub = pltpu.async_copy(ub_hbm.at[uidx_v], ubg, sem)
        cib = pltpu.async_copy(ib_hbm.at[iidx_v], ibg, sem)
        cu.wait()
        ci.wait()
        cub.wait()
        cib.wait()
        pltpu.sync_copy(uw_v, uw_out.at[pl.ds(base, b_per_w)])
        pltpu.sync_copy(iw_v, iw_out.at[pl.ds(base, b_per_w)])
        pltpu.sync_copy(ubg, ub_out.at[pl.ds(base, b_per_w)])
        pltpu.sync_copy(ibg, ib_out.at[pl.ds(base, b_per_w)])

    return gather(user_ids, item_ids, user_weight, ub2d, item_weight, ib2d)


def _tc_rating(uw_g, iw_g, ub_g, ib_g, bias11):
    B, D = uw_g.shape
    BM = 512

    def body(uw_ref, iw_ref, ub_ref, ib_ref, b_ref, out_ref):
        acc = lax.dot_general(
            uw_ref[...], iw_ref[...], (((1,), (1,)), ((), ())),
            preferred_element_type=jnp.float32)
        i = pl.program_id(0)
        ub_blk = ub_ref[pl.ds(i * BM, BM), :]
        out_ref[...] = acc + ub_blk + ib_ref[...] + b_ref[0, 0]

    return pl.pallas_call(
        body,
        grid=(B // BM,),
        in_specs=[
            pl.BlockSpec((BM, D), lambda i: (i, 0)),
            pl.BlockSpec((B, D), lambda i: (0, 0)),
            pl.BlockSpec((B, 1), lambda i: (0, 0)),
            pl.BlockSpec((1, B), lambda i: (0, 0)),
            pl.BlockSpec((1, 1), lambda i: (0, 0)),
        ],
        out_specs=pl.BlockSpec((BM, B), lambda i: (i, 0)),
        out_shape=jax.ShapeDtypeStruct((B, B), jnp.float32),
    )(uw_g, iw_g, ub_g, ib_g, bias11)


def kernel(user_ids, item_ids, user_weight, user_bias, item_weight, item_bias, bias):
    B = user_ids.shape[0]
    uw_g, ub_g, iw_g, ib_g = _sc_gather(user_ids, item_ids, user_weight,
                                        user_bias, item_weight, item_bias)
    ub2 = ub_g
    ib2 = jnp.reshape(ib_g, (1, B))
    bias11 = jnp.reshape(bias, (1, 1))
    return _tc_rating(uw_g, iw_g, ub2, ib2, bias11)


# speedup vs baseline: 13.1589x; 13.1589x over previous
import functools

import jax
import jax.numpy as jnp
from jax import lax
from jax.experimental import pallas as pl
from jax.experimental.pallas import tpu as pltpu
from jax.experimental.pallas import tpu_sc as plsc

_NC = 2
_NS = 16
_NW = _NC * _NS
_GRP = 8
_LANES = 128


def _sc_gather(user_ids, item_ids, uwT, ub1, iwT, ib1):
    B = user_ids.shape[0]
    D = uwT.shape[0]
    b_per_w = B // _NW
    n_grp = b_per_w // _GRP

    mesh = plsc.VectorSubcoreMesh(core_axis_name="c", subcore_axis_name="s")

    @functools.partial(
        pl.kernel,
        mesh=mesh,
        out_type=[
            jax.ShapeDtypeStruct((B, D), jnp.float32),
            jax.ShapeDtypeStruct((B,), jnp.float32),
            jax.ShapeDtypeStruct((B, D), jnp.float32),
            jax.ShapeDtypeStruct((B,), jnp.float32),
        ],
        scratch_types=[
            pltpu.VMEM((b_per_w + 16,), jnp.int32),
            pltpu.VMEM((b_per_w + 16,), jnp.int32),
            pltpu.VMEM((2, _GRP, D, _LANES), jnp.float32),
            pltpu.VMEM((b_per_w, D), jnp.float32),
            pltpu.VMEM((b_per_w, D), jnp.float32),
            pltpu.VMEM((b_per_w,), jnp.float32),
            pltpu.VMEM((b_per_w,), jnp.float32),
            pltpu.SemaphoreType.DMA,
            pltpu.SemaphoreType.DMA,
            pltpu.SemaphoreType.DMA,
        ],
    )
    def gather(uids_hbm, iids_hbm, uwT_hbm, ub1_hbm, iwT_hbm, ib1_hbm,
               uw_out, ub_out, iw_out, ib_out,
               uidx_v, iidx_v, blk, uwr, iwr, ubg, ibg, sem0, sem1, semb):
        wid = lax.axis_index("s") * _NC + lax.axis_index("c")
        base = wid * b_per_w
        pltpu.sync_copy(uids_hbm.at[pl.ds(base, b_per_w)],
                        uidx_v.at[pl.ds(0, b_per_w)])
        pltpu.sync_copy(iids_hbm.at[pl.ds(base, b_per_w)],
                        iidx_v.at[pl.ds(0, b_per_w)])

        cub = pltpu.async_copy(ub1_hbm.at[uidx_v.at[pl.ds(0, b_per_w)]],
                               ubg, semb)
        cib = pltpu.async_copy(ib1_hbm.at[iidx_v.at[pl.ds(0, b_per_w)]],
                               ibg, semb)

        iot = lax.iota(jnp.int32, 16)

        def ids_of(idx_v, g):
            return [idx_v[pl.ds(g * _GRP + j, 16)][0] for j in range(_GRP)]

        def make_table(tbl_hbm, idx_v, rows):
            def issue(g, bset, sem):
                ids = ids_of(idx_v, g)
                for j in range(_GRP):
                    off = pl.multiple_of((ids[j] // _LANES) * _LANES, _LANES)
                    pltpu.async_copy(tbl_hbm.at[:, pl.ds(off, _LANES)],
                                     blk.at[bset, j], sem)

            def drain(bset, sem):
                for j in range(_GRP):
                    pltpu.make_async_copy(tbl_hbm.at[:, pl.ds(0, _LANES)],
                                          blk.at[bset, j], sem).wait()

            def extract(g, bset):
                ids = ids_of(idx_v, g)
                for j in range(_GRP):
                    l = ids[j] - (ids[j] // _LANES) * _LANES
                    col_lo = jnp.zeros((16,), jnp.float32)
                    col_hi = jnp.zeros((16,), jnp.float32)
                    for d in range(16):
                        w = blk[bset, j, d, pl.ds(l, 16)]
                        col_lo = jnp.where(iot == d,
                                           jnp.full((16,), w[0], jnp.float32),
                                           col_lo)
                        w2 = blk[bset, j, d + 16, pl.ds(l, 16)]
                        col_hi = jnp.where(iot == d,
                                           jnp.full((16,), w2[0], jnp.float32),
                                           col_hi)
                    rows[g * _GRP + j, pl.ds(0, 16)] = col_lo
                    rows[g * _GRP + j, pl.ds(16, 16)] = col_hi

            issue(0, 0, sem0)
            issue(1, 1, sem1)

            def body(t, carry):
                g0 = 2 * t
                drain(0, sem0)
                extract(g0, 0)
                issue(g0 + 2, 0, sem0)
                drain(1, sem1)
                extract(g0 + 1, 1)
                issue(g0 + 3, 1, sem1)
                return carry

            lax.fori_loop(0, n_grp // 2 - 1, body, 0)
            gl = n_grp - 2
            drain(0, sem0)
            extract(gl, 0)
            drain(1, sem1)
            extract(gl + 1, 1)

        make_table(uwT_hbm, uidx_v, uwr)
        make_table(iwT_hbm, iidx_v, iwr)

        cub.wait()
        cib.wait()

        pltpu.sync_copy(uwr, uw_out.at[pl.ds(base, b_per_w)])
        pltpu.sync_copy(iwr, iw_out.at[pl.ds(base, b_per_w)])
        pltpu.sync_copy(ubg, ub_out.at[pl.ds(base, b_per_w)])
        pltpu.sync_copy(ibg, ib_out.at[pl.ds(base, b_per_w)])

    return gather(user_ids, item_ids, uwT, ub1, iwT, ib1)


def _tc_rating(uw_g, iw_g, ub_g, ib_g, bias11):
    B, D = uw_g.shape
    BM = 512

    def body(uw_ref, iw_ref, ub_ref, ib_ref, b_ref, out_ref):
        acc = lax.dot_general(
            uw_ref[...], iw_ref[...], (((1,), (1,)), ((), ())),
            preferred_element_type=jnp.float32)
        i = pl.program_id(0)
        ub_blk = ub_ref[pl.ds(i * BM, BM), :]
        out_ref[...] = acc + ub_blk + ib_ref[...] + b_ref[0, 0]

    return pl.pallas_call(
        body,
        grid=(B // BM,),
        in_specs=[
            pl.BlockSpec((BM, D), lambda i: (i, 0)),
            pl.BlockSpec((B, D), lambda i: (0, 0)),
            pl.BlockSpec((B, 1), lambda i: (0, 0)),
            pl.BlockSpec((1, B), lambda i: (0, 0)),
            pl.BlockSpec((1, 1), lambda i: (0, 0)),
        ],
        out_specs=pl.BlockSpec((BM, B), lambda i: (i, 0)),
        out_shape=jax.ShapeDtypeStruct((B, B), jnp.float32),
    )(uw_g, iw_g, ub_g, ib_g, bias11)


def kernel(user_ids, item_ids, user_weight, user_bias, item_weight, item_bias, bias):
    B = user_ids.shape[0]
    uwT = user_weight.T
    iwT = item_weight.T
    ub1 = jnp.reshape(user_bias, (-1,))
    ib1 = jnp.reshape(item_bias, (-1,))
    uw_g, ub_g, iw_g, ib_g = _sc_gather(user_ids, item_ids, uwT, ub1, iwT, ib1)
    ub2 = jnp.reshape(ub_g, (B, 1))
    ib2 = jnp.reshape(ib_g, (1, B))
    bias11 = jnp.reshape(bias, (1, 1))
    return _tc_rating(uw_g, iw_g, ub2, ib2, bias11)


# speedup vs baseline: 20.6377x; 1.5683x over previous
import functools

import jax
import jax.numpy as jnp
from jax import lax
from jax.experimental import pallas as pl
from jax.experimental.pallas import tpu as pltpu
from jax.experimental.pallas import tpu_sc as plsc

_NC = 2
_NS = 16
_NW = _NC * _NS
_GRP = 8
_LANES = 128


def _sc_gather(user_ids, item_ids, uwT, ubT, iwT, ibT):
    B = user_ids.shape[0]
    D = uwT.shape[0]
    b_per_w = B // _NW
    n_grp = b_per_w // _GRP

    mesh = plsc.VectorSubcoreMesh(core_axis_name="c", subcore_axis_name="s")

    @functools.partial(
        pl.kernel,
        mesh=mesh,
        out_type=[
            jax.ShapeDtypeStruct((B, D), jnp.float32),
            jax.ShapeDtypeStruct((B,), jnp.float32),
            jax.ShapeDtypeStruct((B, D), jnp.float32),
            jax.ShapeDtypeStruct((B,), jnp.float32),
        ],
        scratch_types=[
            pltpu.VMEM((b_per_w + 16,), jnp.int32),
            pltpu.VMEM((b_per_w + 16,), jnp.int32),
            pltpu.VMEM((2, _GRP, D, _LANES), jnp.float32),
            pltpu.VMEM((b_per_w, D), jnp.float32),
            pltpu.VMEM((b_per_w, D), jnp.float32),
            pltpu.VMEM((b_per_w + 16,), jnp.float32),
            pltpu.VMEM((b_per_w + 16,), jnp.float32),
            pltpu.VMEM((b_per_w, _LANES), jnp.float32),
            pltpu.SemaphoreType.DMA,
            pltpu.SemaphoreType.DMA,
            pltpu.SemaphoreType.DMA,
        ],
    )
    def gather(uids_hbm, iids_hbm, uwT_hbm, ubT_hbm, iwT_hbm, ibT_hbm,
               uw_out, ub_out, iw_out, ib_out,
               uidx_v, iidx_v, blk, uwr, iwr, ubg, ibg, bb,
               sem0, sem1, semb):
        wid = lax.axis_index("s") * _NC + lax.axis_index("c")
        base = wid * b_per_w
        pltpu.sync_copy(uids_hbm.at[pl.ds(base, b_per_w)],
                        uidx_v.at[pl.ds(0, b_per_w)])
        pltpu.sync_copy(iids_hbm.at[pl.ds(base, b_per_w)],
                        iidx_v.at[pl.ds(0, b_per_w)])

        iot = lax.iota(jnp.int32, 16)

        def ids_of(idx_v, g):
            return [idx_v[pl.ds(g * _GRP + j, 16)][0] for j in range(_GRP)]

        def bias_issue(btbl_hbm, idx_v):
            def step(t, carry):
                u = idx_v[pl.ds(t, 16)][0]
                off = pl.multiple_of((u // _LANES) * _LANES, _LANES)
                pltpu.async_copy(btbl_hbm.at[:, pl.ds(off, _LANES)],
                                 bb.at[pl.ds(t, 1), :], semb)
                return carry
            lax.fori_loop(0, b_per_w, step, 0)

        def bias_finish(btbl_hbm, idx_v, vals):
            def step(t, carry):
                pltpu.make_async_copy(btbl_hbm.at[:, pl.ds(0, _LANES)],
                                      bb.at[pl.ds(0, 1), :], semb).wait()
                return carry
            lax.fori_loop(0, b_per_w, step, 0)
            for g in range(n_grp):
                ids8 = ids_of(idx_v, g)
                val = jnp.zeros((16,), jnp.float32)
                for j in range(_GRP):
                    i = g * _GRP + j
                    l = ids8[j] - (ids8[j] // _LANES) * _LANES
                    w = bb[i, pl.ds(l, 16)]
                    val = jnp.where(iot == j,
                                    jnp.full((16,), w[0], jnp.float32), val)
                vals[pl.ds(g * _GRP, 16)] = val

        def make_table(tbl_hbm, idx_v, rows):
            def issue(g, bset, sem):
                ids = ids_of(idx_v, g)
                for j in range(_GRP):
                    off = pl.multiple_of((ids[j] // _LANES) * _LANES, _LANES)
                    pltpu.async_copy(tbl_hbm.at[:, pl.ds(off, _LANES)],
                                     blk.at[bset, j], sem)

            def drain(bset, sem):
                for j in range(_GRP):
                    pltpu.make_async_copy(tbl_hbm.at[:, pl.ds(0, _LANES)],
                                          blk.at[bset, j], sem).wait()

            def extract(g, bset):
                ids = ids_of(idx_v, g)
                for j in range(_GRP):
                    l = ids[j] - (ids[j] // _LANES) * _LANES
                    col_lo = jnp.zeros((16,), jnp.float32)
                    col_hi = jnp.zeros((16,), jnp.float32)
                    for d in range(16):
                        w = blk[bset, j, d, pl.ds(l, 16)]
                        col_lo = jnp.where(iot == d,
                                           jnp.full((16,), w[0], jnp.float32),
                                           col_lo)
                        w2 = blk[bset, j, d + 16, pl.ds(l, 16)]
                        col_hi = jnp.where(iot == d,
                                           jnp.full((16,), w2[0], jnp.float32),
                                           col_hi)
                    rows[g * _GRP + j, pl.ds(0, 16)] = col_lo
                    rows[g * _GRP + j, pl.ds(16, 16)] = col_hi

            issue(0, 0, sem0)
            issue(1, 1, sem1)

            def body(t, carry):
                g0 = 2 * t
                drain(0, sem0)
                extract(g0, 0)
                issue(g0 + 2, 0, sem0)
                drain(1, sem1)
                extract(g0 + 1, 1)
                issue(g0 + 3, 1, sem1)
                return carry

            lax.fori_loop(0, n_grp // 2 - 1, body, 0)
            gl = n_grp - 2
            drain(0, sem0)
            extract(gl, 0)
            drain(1, sem1)
            extract(gl + 1, 1)

        bias_issue(ubT_hbm, uidx_v)
        make_table(uwT_hbm, uidx_v, uwr)
        bias_finish(ubT_hbm, uidx_v, ubg)
        bias_issue(ibT_hbm, iidx_v)
        make_table(iwT_hbm, iidx_v, iwr)
        bias_finish(ibT_hbm, iidx_v, ibg)

        pltpu.sync_copy(uwr, uw_out.at[pl.ds(base, b_per_w)])
        pltpu.sync_copy(iwr, iw_out.at[pl.ds(base, b_per_w)])
        pltpu.sync_copy(ubg.at[pl.ds(0, b_per_w)],
                        ub_out.at[pl.ds(base, b_per_w)])
        pltpu.sync_copy(ibg.at[pl.ds(0, b_per_w)],
                        ib_out.at[pl.ds(base, b_per_w)])

    return gather(user_ids, item_ids, uwT, ubT, iwT, ibT)


def _tc_rating(uw_g, iw_g, ub_g, ib_g, bias11):
    B, D = uw_g.shape
    BM = 512

    def body(uw_ref, iw_ref, ub_ref, ib_ref, b_ref, out_ref):
        acc = lax.dot_general(
            uw_ref[...], iw_ref[...], (((1,), (1,)), ((), ())),
            preferred_element_type=jnp.float32)
        i = pl.program_id(0)
        ub_blk = ub_ref[pl.ds(i * BM, BM), :]
        out_ref[...] = acc + ub_blk + ib_ref[...] + b_ref[0, 0]

    return pl.pallas_call(
        body,
        grid=(B // BM,),
        in_specs=[
            pl.BlockSpec((BM, D), lambda i: (i, 0)),
            pl.BlockSpec((B, D), lambda i: (0, 0)),
            pl.BlockSpec((B, 1), lambda i: (0, 0)),
            pl.BlockSpec((1, B), lambda i: (0, 0)),
            pl.BlockSpec((1, 1), lambda i: (0, 0)),
        ],
        out_specs=pl.BlockSpec((BM, B), lambda i: (i, 0)),
        out_shape=jax.ShapeDtypeStruct((B, B), jnp.float32),
    )(uw_g, iw_g, ub_g, ib_g, bias11)


def kernel(user_ids, item_ids, user_weight, user_bias, item_weight, item_bias, bias):
    B = user_ids.shape[0]
    uwT = user_weight.T
    iwT = item_weight.T
    ubT = user_bias.T
    ibT = item_bias.T
    uw_g, ub_g, iw_g, ib_g = _sc_gather(user_ids, item_ids, uwT, ubT, iwT, ibT)
    ub2 = jnp.reshape(ub_g, (B, 1))
    ib2 = jnp.reshape(ib_g, (1, B))
    bias11 = jnp.reshape(bias, (1, 1))
    return _tc_rating(uw_g, iw_g, ub2, ib2, bias11)


# speedup vs baseline: 21.2114x; 1.0278x over previous
import functools

import jax
import jax.numpy as jnp
from jax import lax
from jax.experimental import pallas as pl
from jax.experimental.pallas import tpu as pltpu
from jax.experimental.pallas import tpu_sc as plsc

_NC = 2
_NS = 16
_NW = _NC * _NS
_GRP = 8
_LANES = 128


def _sc_gather(user_ids, item_ids, uwT, ubT, iwT, ibT):
    B = user_ids.shape[0]
    D = uwT.shape[0]
    b_per_w = B // _NW
    n_grp = b_per_w // _GRP

    mesh = plsc.VectorSubcoreMesh(core_axis_name="c", subcore_axis_name="s")

    @functools.partial(
        pl.kernel,
        mesh=mesh,
        out_type=[
            jax.ShapeDtypeStruct((B, D), jnp.float32),
            jax.ShapeDtypeStruct((B,), jnp.float32),
            jax.ShapeDtypeStruct((B, D), jnp.float32),
            jax.ShapeDtypeStruct((B,), jnp.float32),
        ],
        scratch_types=[
            pltpu.VMEM((b_per_w + 16,), jnp.int32),
            pltpu.VMEM((b_per_w + 16,), jnp.int32),
            pltpu.VMEM((2, _GRP, D, _LANES), jnp.float32),
            pltpu.VMEM((b_per_w, D), jnp.float32),
            pltpu.VMEM((b_per_w, D), jnp.float32),
            pltpu.VMEM((b_per_w + 16,), jnp.float32),
            pltpu.VMEM((b_per_w + 16,), jnp.float32),
            pltpu.VMEM((b_per_w, _LANES), jnp.float32),
            pltpu.SemaphoreType.DMA,
            pltpu.SemaphoreType.DMA,
            pltpu.SemaphoreType.DMA,
        ],
    )
    def gather(uids_hbm, iids_hbm, uwT_hbm, ubT_hbm, iwT_hbm, ibT_hbm,
               uw_out, ub_out, iw_out, ib_out,
               uidx_v, iidx_v, blk, uwr, iwr, ubg, ibg, bb,
               sem0, sem1, semb):
        wid = lax.axis_index("s") * _NC + lax.axis_index("c")
        base = wid * b_per_w
        pltpu.sync_copy(uids_hbm.at[pl.ds(base, b_per_w)],
                        uidx_v.at[pl.ds(0, b_per_w)])
        pltpu.sync_copy(iids_hbm.at[pl.ds(base, b_per_w)],
                        iidx_v.at[pl.ds(0, b_per_w)])

        iot = lax.iota(jnp.int32, 16)

        def ids_of(idx_v, g):
            vec = idx_v[pl.ds(g * _GRP, 16)]
            return [vec[j] for j in range(_GRP)]

        def bias_issue(btbl_hbm, idx_v):
            def step(g, carry):
                ids8 = ids_of(idx_v, g)
                for j in range(_GRP):
                    off = pl.multiple_of((ids8[j] // _LANES) * _LANES,
                                         _LANES)
                    pltpu.async_copy(btbl_hbm.at[:, pl.ds(off, _LANES)],
                                     bb.at[pl.ds(g * _GRP + j, 1), :], semb)
                return carry
            lax.fori_loop(0, n_grp, step, 0)

        def bias_finish(btbl_hbm, idx_v, vals):
            def step(t, carry):
                for _ in range(_GRP):
                    pltpu.make_async_copy(btbl_hbm.at[:, pl.ds(0, _LANES)],
                                          bb.at[pl.ds(0, 1), :], semb).wait()
                return carry
            lax.fori_loop(0, n_grp, step, 0)
            for g in range(n_grp):
                ids8 = ids_of(idx_v, g)
                val = jnp.zeros((16,), jnp.float32)
                for j in range(_GRP):
                    i = g * _GRP + j
                    l = ids8[j] - (ids8[j] // _LANES) * _LANES
                    w = bb[i, pl.ds(l, 16)]
                    val = jnp.where(iot == j,
                                    jnp.full((16,), w[0], jnp.float32), val)
                vals[pl.ds(g * _GRP, 16)] = val

        def make_table(tbl_hbm, idx_v, rows):
            def issue(g, bset, sem):
                ids = ids_of(idx_v, g)
                for j in range(_GRP):
                    off = pl.multiple_of((ids[j] // _LANES) * _LANES, _LANES)
                    pltpu.async_copy(tbl_hbm.at[:, pl.ds(off, _LANES)],
                                     blk.at[bset, j], sem)

            def drain(bset, sem):
                for j in range(_GRP):
                    pltpu.make_async_copy(tbl_hbm.at[:, pl.ds(0, _LANES)],
                                          blk.at[bset, j], sem).wait()

            def extract(g, bset):
                ids = ids_of(idx_v, g)
                for j in range(_GRP):
                    l = ids[j] - (ids[j] // _LANES) * _LANES
                    col_lo = jnp.zeros((16,), jnp.float32)
                    col_hi = jnp.zeros((16,), jnp.float32)
                    for d in range(16):
                        w = blk[bset, j, d, pl.ds(l, 16)]
                        col_lo = jnp.where(iot == d,
                                           jnp.full((16,), w[0], jnp.float32),
                                           col_lo)
                        w2 = blk[bset, j, d + 16, pl.ds(l, 16)]
                        col_hi = jnp.where(iot == d,
                                           jnp.full((16,), w2[0], jnp.float32),
                                           col_hi)
                    rows[g * _GRP + j, pl.ds(0, 16)] = col_lo
                    rows[g * _GRP + j, pl.ds(16, 16)] = col_hi

            issue(0, 0, sem0)
            issue(1, 1, sem1)

            def body(t, carry):
                g0 = 2 * t
                drain(0, sem0)
                extract(g0, 0)
                issue(g0 + 2, 0, sem0)
                drain(1, sem1)
                extract(g0 + 1, 1)
                issue(g0 + 3, 1, sem1)
                return carry

            lax.fori_loop(0, n_grp // 2 - 1, body, 0)
            gl = n_grp - 2
            drain(0, sem0)
            extract(gl, 0)
            drain(1, sem1)
            extract(gl + 1, 1)

        bias_issue(ubT_hbm, uidx_v)
        make_table(uwT_hbm, uidx_v, uwr)
        bias_finish(ubT_hbm, uidx_v, ubg)
        bias_issue(ibT_hbm, iidx_v)
        make_table(iwT_hbm, iidx_v, iwr)
        bias_finish(ibT_hbm, iidx_v, ibg)

        pltpu.sync_copy(uwr, uw_out.at[pl.ds(base, b_per_w)])
        pltpu.sync_copy(iwr, iw_out.at[pl.ds(base, b_per_w)])
        pltpu.sync_copy(ubg.at[pl.ds(0, b_per_w)],
                        ub_out.at[pl.ds(base, b_per_w)])
        pltpu.sync_copy(ibg.at[pl.ds(0, b_per_w)],
                        ib_out.at[pl.ds(base, b_per_w)])

    return gather(user_ids, item_ids, uwT, ubT, iwT, ibT)


def _tc_rating(uw_g, iw_g, ub_g, ib_g, bias11):
    B, D = uw_g.shape
    BM = 512

    def body(uw_ref, iw_ref, ub_ref, ib_ref, b_ref, out_ref):
        acc = lax.dot_general(
            uw_ref[...], iw_ref[...], (((1,), (1,)), ((), ())),
            preferred_element_type=jnp.float32)
        i = pl.program_id(0)
        ub_blk = ub_ref[pl.ds(i * BM, BM), :]
        out_ref[...] = acc + ub_blk + ib_ref[...] + b_ref[0, 0]

    return pl.pallas_call(
        body,
        grid=(B // BM,),
        in_specs=[
            pl.BlockSpec((BM, D), lambda i: (i, 0)),
            pl.BlockSpec((B, D), lambda i: (0, 0)),
            pl.BlockSpec((B, 1), lambda i: (0, 0)),
            pl.BlockSpec((1, B), lambda i: (0, 0)),
            pl.BlockSpec((1, 1), lambda i: (0, 0)),
        ],
        out_specs=pl.BlockSpec((BM, B), lambda i: (i, 0)),
        out_shape=jax.ShapeDtypeStruct((B, B), jnp.float32),
    )(uw_g, iw_g, ub_g, ib_g, bias11)


def kernel(user_ids, item_ids, user_weight, user_bias, item_weight, item_bias, bias):
    B = user_ids.shape[0]
    uwT = user_weight.T
    iwT = item_weight.T
    ubT = user_bias.T
    ibT = item_bias.T
    uw_g, ub_g, iw_g, ib_g = _sc_gather(user_ids, item_ids, uwT, ubT, iwT, ibT)
    ub2 = jnp.reshape(ub_g, (B, 1))
    ib2 = jnp.reshape(ib_g, (1, B))
    bias11 = jnp.reshape(bias, (1, 1))
    return _tc_rating(uw_g, iw_g, ub2, ib2, bias11)
